# Initial kernel scaffold; baseline (speedup 1.0000x reference)
import jax
import jax.numpy as jnp
from jax.experimental import pallas as pl


def kernel(rotMat, output_rotmats):
    raise NotImplementedError("write your pallas kernel here")


# speedup vs baseline: 1.3480x; 1.3480x over previous
import functools

import jax
import jax.numpy as jnp
from jax import lax
from jax.experimental import pallas as pl
from jax.experimental.pallas import tpu as pltpu
from jax.experimental.pallas import tpu_sc as plsc

B = 4096
P = 36864
PT = 512
NP = P // PT

SC_CORES = 2
SC_SUBCORES = 16
NW = SC_CORES * SC_SUBCORES
B_PER_W = B // NW


def _argmax_body(a_ref, t_ref, best_ref, idx_ref):
    j = pl.program_id(0)
    s = jnp.dot(a_ref[...], t_ref[...], preferred_element_type=jnp.float32)
    m = jnp.max(s, axis=1, keepdims=True)
    col = lax.broadcasted_iota(jnp.int32, s.shape, 1)
    loc = jnp.min(jnp.where(s == m, col, PT), axis=1, keepdims=True) + j * PT

    @pl.when(j == 0)
    def _():
        best_ref[...] = m
        idx_ref[...] = loc

    @pl.when(j > 0)
    def _():
        prev = best_ref[...]
        upd = m > prev
        best_ref[...] = jnp.where(upd, m, prev)
        idx_ref[...] = jnp.where(upd, loc, idx_ref[...])


def _sc_gather(table_pad, idxs):
    mesh = plsc.VectorSubcoreMesh(core_axis_name="c", subcore_axis_name="s")

    @functools.partial(
        pl.kernel,
        mesh=mesh,
        out_type=jax.ShapeDtypeStruct((B, 16), jnp.float32),
        scratch_types=[
            pltpu.VMEM((B_PER_W,), jnp.int32),
            pltpu.VMEM((B_PER_W, 16), jnp.float32),
            pltpu.SemaphoreType.DMA,
        ],
        compiler_params=pltpu.CompilerParams(use_tc_tiling_on_sc=False),
    )
    def gather_k(table_hbm, idx_hbm, out_hbm, idx_v, rows_v, sem):
        wid = lax.axis_index("s") * SC_CORES + lax.axis_index("c")
        base = wid * B_PER_W
        pltpu.sync_copy(idx_hbm.at[pl.ds(base, B_PER_W)], idx_v)
        pltpu.async_copy(table_hbm.at[idx_v], rows_v, sem).wait()
        pltpu.sync_copy(rows_v, out_hbm.at[pl.ds(base, B_PER_W)])

    return gather_k(table_pad, idxs)


def kernel(rotMat, output_rotmats):
    a = rotMat.reshape(B, 9)
    t = output_rotmats.reshape(P, 9)
    tt = t.T

    best, idx = pl.pallas_call(
        _argmax_body,
        grid=(NP,),
        in_specs=[
            pl.BlockSpec((B, 9), lambda j: (0, 0)),
            pl.BlockSpec((9, PT), lambda j: (0, j)),
        ],
        out_specs=[
            pl.BlockSpec((B, 1), lambda j: (0, 0)),
            pl.BlockSpec((B, 1), lambda j: (0, 0)),
        ],
        out_shape=[
            jax.ShapeDtypeStruct((B, 1), jnp.float32),
            jax.ShapeDtypeStruct((B, 1), jnp.int32),
        ],
    )(a, tt)

    table_pad = jnp.pad(t, ((0, 0), (0, 7)))
    rows = _sc_gather(table_pad, idx.reshape(B))
    nearest = rows[:, :9].reshape(B, 3, 3)
    return best.reshape(B), nearest
